# baseline (speedup 1.0000x reference)
import functools

import jax
import jax.numpy as jnp
from jax import lax
from jax.experimental import pallas as pl
from jax.experimental.pallas import tpu as pltpu
from jax.experimental.pallas import tpu_sc as plsc

_B = 16384
_D = 64
_NC = 2
_NS = 16
_NW = _NC * _NS
_BPW = _B // _NW
_CH = 128
_NCHUNK = _BPW // _CH

@functools.cache
def _make_sc_gather():
    mesh = plsc.VectorSubcoreMesh(core_axis_name="c", subcore_axis_name="s")

    @functools.partial(
        pl.kernel,
        mesh=mesh,
        out_type=[
            jax.ShapeDtypeStruct((_B, _D), jnp.float32),
            jax.ShapeDtypeStruct((_B, _D), jnp.float32),
        ],
        scratch_types=[
            pltpu.VMEM((_NCHUNK, _CH), jnp.int32),
            pltpu.VMEM((_NCHUNK, _CH), jnp.int32),
            pltpu.VMEM((_BPW, _D), jnp.float32),
            pltpu.VMEM((_BPW, _D), jnp.float32),
            pltpu.SemaphoreType.DMA,
            pltpu.SemaphoreType.DMA,
        ],
        compiler_params=pltpu.CompilerParams(use_tc_tiling_on_sc=False),
    )
    def _sc_gather(uidx_hbm, midx_hbm, utab_hbm, mtab_hbm,
                   uout_hbm, mout_hbm,
                   uidx_v, midx_v, urows_v, mrows_v, usem, msem):
        wid = lax.axis_index("s") * _NC + lax.axis_index("c")
        base = wid * _BPW
        pltpu.sync_copy(uidx_hbm.at[pl.ds(wid * _NCHUNK, _NCHUNK)], uidx_v)
        pltpu.sync_copy(midx_hbm.at[pl.ds(wid * _NCHUNK, _NCHUNK)], midx_v)
        copies = []
        for j in range(_NCHUNK):
            copies.append(pltpu.async_copy(
                utab_hbm.at[uidx_v.at[j]], urows_v.at[pl.ds(j * _CH, _CH)], usem))
            copies.append(pltpu.async_copy(
                mtab_hbm.at[midx_v.at[j]], mrows_v.at[pl.ds(j * _CH, _CH)], msem))
        for c in copies:
            c.wait()
        pltpu.sync_copy(urows_v, uout_hbm.at[pl.ds(base, _BPW)])
        pltpu.sync_copy(mrows_v, mout_hbm.at[pl.ds(base, _BPW)])

    return _sc_gather


def _mlp_body(u_ref, m_ref, w1_ref, b1_ref, w2t_ref, b2_ref, o_ref):
    w1 = w1_ref[...]
    h = jnp.dot(u_ref[...], w1[:_D], preferred_element_type=jnp.float32)
    h = h + jnp.dot(m_ref[...], w1[_D:], preferred_element_type=jnp.float32)
    h = jnp.maximum(h + b1_ref[...], 0.0)
    o_ref[...] = jnp.sum(h * w2t_ref[...], axis=1, keepdims=True) + b2_ref[...]


_MLP_BS = 4096


def _mlp(u, m, w1, b1, w2t, b2):
    grid = (_B // _MLP_BS,)
    return pl.pallas_call(
        _mlp_body,
        grid=grid,
        in_specs=[
            pl.BlockSpec((_MLP_BS, _D), lambda i: (i, 0)),
            pl.BlockSpec((_MLP_BS, _D), lambda i: (i, 0)),
            pl.BlockSpec((2 * _D, _D), lambda i: (0, 0)),
            pl.BlockSpec((1, _D), lambda i: (0, 0)),
            pl.BlockSpec((1, _D), lambda i: (0, 0)),
            pl.BlockSpec((1, 1), lambda i: (0, 0)),
        ],
        out_specs=pl.BlockSpec((_MLP_BS, 1), lambda i: (i, 0)),
        out_shape=jax.ShapeDtypeStruct((_B, 1), jnp.float32),
    )(u, m, w1, b1, w2t, b2)


def kernel(user_id, movie_id, user_table, movie_table, W1, b1, W2, b2):
    uidx = user_id.astype(jnp.int32).reshape(_NW * _NCHUNK, _CH)
    midx = movie_id.astype(jnp.int32).reshape(_NW * _NCHUNK, _CH)
    u, m = _make_sc_gather()(uidx, midx, user_table, movie_table)
    return _mlp(u, m, W1, b1.reshape(1, _D), W2.reshape(1, _D), b2.reshape(1, 1))


# speedup vs baseline: 1.5748x; 1.5748x over previous
import functools

import jax
import jax.numpy as jnp
from jax import lax
from jax.experimental import pallas as pl
from jax.experimental.pallas import tpu as pltpu
from jax.experimental.pallas import tpu_sc as plsc

_B = 16384
_D = 64
_NC = 2
_NS = 16
_NW = _NC * _NS
_BPW = _B // _NW
_CH = 256
_NCHUNK = _BPW // _CH

@functools.cache
def _make_sc_gather():
    mesh = plsc.VectorSubcoreMesh(core_axis_name="c", subcore_axis_name="s")

    @functools.partial(
        pl.kernel,
        mesh=mesh,
        out_type=[
            jax.ShapeDtypeStruct((_B, _D), jnp.float32),
            jax.ShapeDtypeStruct((_B, _D), jnp.float32),
        ],
        scratch_types=[
            pltpu.VMEM((_BPW,), jnp.int32),
            pltpu.VMEM((_BPW,), jnp.int32),
            pltpu.VMEM((_CH, _D), jnp.float32),
            pltpu.VMEM((_CH, _D), jnp.float32),
            pltpu.SemaphoreType.DMA,
            pltpu.SemaphoreType.DMA,
        ],
        compiler_params=pltpu.CompilerParams(use_tc_tiling_on_sc=True),
    )
    def _sc_gather(uidx_hbm, midx_hbm, utab_hbm, mtab_hbm,
                   uout_hbm, mout_hbm,
                   uidx_s, midx_s, urows_v, mrows_v, usem, msem):
        wid = lax.axis_index("s") * _NC + lax.axis_index("c")
        base = wid * _BPW
        pltpu.sync_copy(uidx_hbm.at[pl.ds(base, _BPW)], uidx_s)
        pltpu.sync_copy(midx_hbm.at[pl.ds(base, _BPW)], midx_s)

        for c in range(_NCHUNK):
            coff = c * _CH

            def body(g, carry):
                src_i = coff + g * 16
                dst_i = g * 16
                uvec = uidx_s[pl.ds(src_i, 16)]
                mvec = midx_s[pl.ds(src_i, 16)]
                for j in range(16):
                    pltpu.async_copy(
                        utab_hbm.at[pl.ds(uvec[j], 1)],
                        urows_v.at[pl.ds(dst_i + j, 1)], usem)
                    pltpu.async_copy(
                        mtab_hbm.at[pl.ds(mvec[j], 1)],
                        mrows_v.at[pl.ds(dst_i + j, 1)], msem)
                return carry

            lax.fori_loop(0, _CH // 16, body, 0)
            pltpu.make_async_copy(
                utab_hbm.at[pl.ds(0, _CH)], urows_v, usem).wait()
            pltpu.make_async_copy(
                mtab_hbm.at[pl.ds(0, _CH)], mrows_v, msem).wait()
            pltpu.sync_copy(urows_v, uout_hbm.at[pl.ds(base + coff, _CH)])
            pltpu.sync_copy(mrows_v, mout_hbm.at[pl.ds(base + coff, _CH)])

    return _sc_gather


def _mlp_body(u_ref, m_ref, w1_ref, b1_ref, w2t_ref, b2_ref, o_ref):
    w1 = w1_ref[...]
    h = jnp.dot(u_ref[...], w1[:_D], preferred_element_type=jnp.float32)
    h = h + jnp.dot(m_ref[...], w1[_D:], preferred_element_type=jnp.float32)
    h = jnp.maximum(h + b1_ref[...], 0.0)
    o_ref[...] = jnp.sum(h * w2t_ref[...], axis=1, keepdims=True) + b2_ref[...]


_MLP_BS = 4096


def _mlp(u, m, w1, b1, w2t, b2):
    grid = (_B // _MLP_BS,)
    return pl.pallas_call(
        _mlp_body,
        grid=grid,
        in_specs=[
            pl.BlockSpec((_MLP_BS, _D), lambda i: (i, 0)),
            pl.BlockSpec((_MLP_BS, _D), lambda i: (i, 0)),
            pl.BlockSpec((2 * _D, _D), lambda i: (0, 0)),
            pl.BlockSpec((1, _D), lambda i: (0, 0)),
            pl.BlockSpec((1, _D), lambda i: (0, 0)),
            pl.BlockSpec((1, 1), lambda i: (0, 0)),
        ],
        out_specs=pl.BlockSpec((_MLP_BS, 1), lambda i: (i, 0)),
        out_shape=jax.ShapeDtypeStruct((_B, 1), jnp.float32),
    )(u, m, w1, b1, w2t, b2)


def kernel(user_id, movie_id, user_table, movie_table, W1, b1, W2, b2):
    uidx = user_id.astype(jnp.int32)
    midx = movie_id.astype(jnp.int32)
    u, m = _make_sc_gather()(uidx, midx, user_table, movie_table)
    return _mlp(u, m, W1, b1.reshape(1, _D), W2.reshape(1, _D), b2.reshape(1, 1))
